# baseline (speedup 1.0000x reference)
import jax
import jax.numpy as jnp
from jax import lax
from jax.experimental import pallas as pl
from jax.experimental.pallas import tpu as pltpu
from jax.experimental.pallas import tpu_sc as plsc

N = 10000
NP = 10240
D = 128
E = 320000
NC = 2
NS = 16
NW = NC * NS
EP = E // NW
ROWS = NP // NS
BN = 1024
CH = 32
NCHUNK = EP // CH
TAIL = EP - NCHUNK * CH
CHE = 16
SEG = 2000


def _mesh():
    return plsc.VectorSubcoreMesh(
        core_axis_name="c", subcore_axis_name="s", num_cores=NC, num_subcores=NS
    )


def _prep_body(x_ref, win_ref, b_ref, w1_ref, w1p_ref, att2_ref,
               hb_ref, hp_ref, asrc_ref, adst_ref):
    t = lax.dot_general(x_ref[...], win_ref[...], (((1,), (1,)), ((), ())),
                        preferred_element_type=jnp.float32) + b_ref[...]
    h = lax.dot_general(t, w1_ref[...], (((1,), (1,)), ((), ())),
                        preferred_element_type=jnp.float32)
    hb_ref[...] = h.astype(jnp.bfloat16)
    hp_ref[...] = lax.dot_general(t, w1p_ref[...], (((1,), (1,)), ((), ())),
                                  preferred_element_type=jnp.float32)
    att2 = att2_ref[...]
    asrc_ref[...] = lax.dot_general(h, att2[:, 0], (((1,), (0,)), ((), ())),
                                    preferred_element_type=jnp.float32)
    adst_ref[...] = lax.dot_general(h, att2[:, 1], (((1,), (0,)), ((), ())),
                                    preferred_element_type=jnp.float32)


def _prep_call(x, w_in, b_in, w1, w1p, att2):
    return pl.pallas_call(
        _prep_body,
        grid=(NP // BN,),
        in_specs=[
            pl.BlockSpec((BN, D), lambda i: (i, 0)),
            pl.BlockSpec((D, D), lambda i: (0, 0)),
            pl.BlockSpec((1, D), lambda i: (0, 0)),
            pl.BlockSpec((D, D), lambda i: (0, 0)),
            pl.BlockSpec((D, D), lambda i: (0, 0)),
            pl.BlockSpec((D, 2), lambda i: (0, 0)),
        ],
        out_specs=[
            pl.BlockSpec((BN, D), lambda i: (i, 0)),
            pl.BlockSpec((BN, D), lambda i: (i, 0)),
            pl.BlockSpec((BN,), lambda i: (i,)),
            pl.BlockSpec((BN,), lambda i: (i,)),
        ],
        out_shape=[
            jax.ShapeDtypeStruct((NP, D), jnp.bfloat16),
            jax.ShapeDtypeStruct((NP, D), jnp.float32),
            jax.ShapeDtypeStruct((NP,), jnp.float32),
            jax.ShapeDtypeStruct((NP,), jnp.float32),
        ],
    )(x, w_in, b_in, w1, w1p, att2)


def _edge_scale(src_v, dst_v, asrc_v, adst_v, s_loc, hbuf, obuf, off):
    o2 = pl.ds(off, 16)
    d16 = dst_v[o2]
    s16 = src_v[o2]
    av = plsc.load_gather(asrc_v, [s16])
    bv = plsc.load_gather(adst_v, [d16])
    e = av + bv
    e = jnp.where(e >= 0, e, 0.2 * e)
    ex = jnp.exp(e)
    plsc.addupdate_scatter(s_loc, [d16], ex)
    for j in range(16):
        exs = ex[j]
        for w in range(4):
            a2 = plsc.bitcast(hbuf[j, pl.ds(w * 16, 16)], jnp.bfloat16)
            he, ho = plsc.unpack(a2, format=plsc.PackFormat.INTERLEAVED)
            obuf[j, pl.ds(w * 32, 16)] = he * exs
            obuf[j, pl.ds(w * 32 + 16, 16)] = ho * exs
    return d16


def _edge_body(asrch, adsth, srch, dsth, hh, s_out, u_out,
               asrc_v, adst_v, src_v, dst_v, s_loc, hbuf0, hbuf1,
               obuf0, obuf1, u_sh, sem0, sem1, sems0, sems1):
    cid = lax.axis_index("c")
    sid = lax.axis_index("s")
    wid = sid * NC + cid
    base = wid * EP

    pltpu.sync_copy(asrch.at[pl.ds(0, N)], asrc_v)
    pltpu.sync_copy(adsth.at[pl.ds(0, N)], adst_v)

    zf = jnp.zeros((16,), jnp.float32)

    def zloop(i, carry):
        s_loc[pl.ds(i * 16, 16)] = zf
        return carry

    lax.fori_loop(0, N // 16, zloop, 0)

    for r in range(CHE):
        for q in range(8):
            obuf0[r, pl.ds(q * 16, 16)] = zf

    def zuloop(i, carry):
        pltpu.sync_copy(obuf0, u_sh.at[pl.ds(sid * ROWS + i * CHE, CHE)])
        return carry

    lax.fori_loop(0, ROWS // CHE, zuloop, 0)
    plsc.subcore_barrier()

    zidx = jnp.zeros((16,), jnp.int32)

    def wait_gather(off, buf, sem):
        pltpu.make_async_copy(hh.at[src_v.at[pl.ds(off, CHE)]],
                              buf, sem).wait()

    def wait_scatter(buf, sem):
        pltpu.make_async_copy(buf, u_sh.at[zidx], sem).wait()


    def seg_loop(si, carry):
        sbase = base + si * SEG
        pltpu.sync_copy(srch.at[pl.ds(sbase, SEG)], src_v)
        pltpu.sync_copy(dsth.at[pl.ds(sbase, SEG)], dst_v)
        pltpu.async_copy(hh.at[src_v.at[pl.ds(0, CHE)]], hbuf0, sem0)

        def pair(io, c2):
            offa = io * (2 * CHE)
            offb = offa + CHE
            offc = offb + CHE
            pltpu.async_copy(hh.at[src_v.at[pl.ds(offb, CHE)]], hbuf1, sem1)
            wait_gather(offa, hbuf0, sem0)

            @pl.when(io > 0)
            def _():
                wait_scatter(obuf0, sems0)

            d16a = _edge_scale(src_v, dst_v, asrc_v, adst_v, s_loc, hbuf0,
                               obuf0, offa)
            pltpu.async_copy(obuf0, u_sh.at[d16a], sems0, add=True)
            pltpu.async_copy(hh.at[src_v.at[pl.ds(offc, CHE)]], hbuf0, sem0)
            wait_gather(offb, hbuf1, sem1)

            @pl.when(io > 0)
            def _():
                wait_scatter(obuf1, sems1)

            d16b = _edge_scale(src_v, dst_v, asrc_v, adst_v, s_loc, hbuf1,
                               obuf1, offb)
            pltpu.async_copy(obuf1, u_sh.at[d16b], sems1, add=True)
            return c2

        lax.fori_loop(0, (SEG // CHE) // 2, pair, 0)
        offe = SEG - CHE
        wait_gather(offe, hbuf0, sem0)
        wait_scatter(obuf0, sems0)
        d16e = _edge_scale(src_v, dst_v, asrc_v, adst_v, s_loc, hbuf0,
                           obuf0, offe)
        wait_scatter(obuf1, sems1)
        pltpu.sync_copy(obuf0, u_sh.at[d16e], add=True)
        return carry

    lax.fori_loop(0, EP // SEG, seg_loop, 0)

    pltpu.sync_copy(s_loc, s_out.at[pl.ds(wid * NP, N)])
    plsc.subcore_barrier()
    pltpu.sync_copy(u_sh.at[pl.ds(sid * ROWS, ROWS)],
                    u_out.at[cid, pl.ds(sid * ROWS, ROWS)])


def _edge_call(asrc, adst, src, dst, h):
    return pl.kernel(
        _edge_body,
        out_type=(
            jax.ShapeDtypeStruct((NW * NP,), jnp.float32),
            jax.ShapeDtypeStruct((NC, NP, D), jnp.float32),
        ),
        mesh=_mesh(),
        compiler_params=pltpu.CompilerParams(needs_layout_passes=False,
                                             use_tc_tiling_on_sc=False),
        scratch_types=[
            pltpu.VMEM((N,), jnp.float32),
            pltpu.VMEM((N,), jnp.float32),
            pltpu.VMEM((SEG,), jnp.int32),
            pltpu.VMEM((SEG,), jnp.int32),
            pltpu.VMEM((N,), jnp.float32),
            pltpu.VMEM((CHE, D // 2), jnp.int32),
            pltpu.VMEM((CHE, D // 2), jnp.int32),
            pltpu.VMEM((CHE, D), jnp.float32),
            pltpu.VMEM((CHE, D), jnp.float32),
            pltpu.VMEM_SHARED((NP, D), jnp.float32),
            pltpu.SemaphoreType.DMA,
            pltpu.SemaphoreType.DMA,
            pltpu.SemaphoreType.DMA,
            pltpu.SemaphoreType.DMA,
        ],
    )(asrc, adst, src, dst, h)


def _comb_body(s_ref, asrc_ref, adst_ref, u_ref, h_ref, b_ref, out_ref):
    sv = jnp.sum(s_ref[...], axis=0)
    al = asrc_ref[...] + adst_ref[...]
    al = jnp.where(al >= 0, al, 0.2 * al)
    al = jnp.exp(al)
    r = 1.0 / (sv + al + 1e-16)
    out_ref[...] = ((u_ref[0] + u_ref[1] + al[:, None] * h_ref[...])
                    * r[:, None] + b_ref[...]).astype(jnp.bfloat16)


def _comb_call(s_part, asrc, adst, u_part, h, bias):
    return pl.pallas_call(
        _comb_body,
        grid=(NP // BN,),
        in_specs=[
            pl.BlockSpec((NW, BN), lambda i: (0, i)),
            pl.BlockSpec((BN,), lambda i: (i,)),
            pl.BlockSpec((BN,), lambda i: (i,)),
            pl.BlockSpec((NC, BN, D), lambda i: (0, i, 0)),
            pl.BlockSpec((BN, D), lambda i: (i, 0)),
            pl.BlockSpec((1, D), lambda i: (0, 0)),
        ],
        out_specs=pl.BlockSpec((BN, D), lambda i: (i, 0)),
        out_shape=jax.ShapeDtypeStruct((NP, D), jnp.bfloat16),
    )(s_part, asrc, adst, u_part, h, bias)


def _edge_dot(abuf, bbuf, row):
    acc = None
    for w in range(4):
        cs = pl.ds(w * 16, 16)
        a2 = plsc.bitcast(abuf[row, cs], jnp.bfloat16)
        b2 = plsc.bitcast(bbuf[row, cs], jnp.bfloat16)
        ae, ao = plsc.unpack(a2, format=plsc.PackFormat.INTERLEAVED)
        be, bo = plsc.unpack(b2, format=plsc.PackFormat.INTERLEAVED)
        t = ae * be + ao * bo
        acc = t if acc is None else acc + t
    return acc


def _dot_chunk(abuf, bbuf, lg_v, off, n_edges):
    lane = lax.iota(jnp.int32, 16)
    for g in range(n_edges // 16):
        dots = jnp.zeros((16,), jnp.float32)
        for j in range(16):
            acc = _edge_dot(abuf, bbuf, g * 16 + j)
            dots = jnp.where(lane == j, jnp.sum(acc), dots)
        lg_v[pl.ds(off + g * 16, 16)] = dots


def _logits_body(outh, e0h, e1h, lg,
                 ia_v, ib_v, abuf0, bbuf0, abuf1, bbuf1, lg_v,
                 sema0, semb0, sema1, semb1):
    cid = lax.axis_index("c")
    sid = lax.axis_index("s")
    wid = sid * NC + cid
    base = wid * EP
    pltpu.sync_copy(e0h.at[pl.ds(base, EP)], ia_v)
    pltpu.sync_copy(e1h.at[pl.ds(base, EP)], ib_v)

    def issue(off, ab, bb, sa, sb):
        pltpu.async_copy(outh.at[ia_v.at[pl.ds(off, CH)]], ab, sa)
        pltpu.async_copy(outh.at[ib_v.at[pl.ds(off, CH)]], bb, sb)

    def drain(off, ab, bb, sa, sb):
        pltpu.make_async_copy(outh.at[ia_v.at[pl.ds(off, CH)]], ab, sa).wait()
        pltpu.make_async_copy(outh.at[ib_v.at[pl.ds(off, CH)]], bb, sb).wait()

    issue(0, abuf0, bbuf0, sema0, semb0)

    def pair(io, carry):
        offa = io * (2 * CH)
        offb = offa + CH
        offc = offb + CH
        issue(offb, abuf1, bbuf1, sema1, semb1)
        drain(offa, abuf0, bbuf0, sema0, semb0)
        _dot_chunk(abuf0, bbuf0, lg_v, offa, CH)

        @pl.when(offc + CH <= EP - TAIL)
        def _():
            issue(offc, abuf0, bbuf0, sema0, semb0)

        drain(offb, abuf1, bbuf1, sema1, semb1)
        _dot_chunk(abuf1, bbuf1, lg_v, offb, CH)
        return carry

    lax.fori_loop(0, NCHUNK // 2, pair, 0)
    if TAIL:
        off = NCHUNK * CH
        ca = pltpu.async_copy(outh.at[ia_v.at[pl.ds(off, TAIL)]],
                              abuf0.at[pl.ds(0, TAIL)], sema0)
        cb = pltpu.async_copy(outh.at[ib_v.at[pl.ds(off, TAIL)]],
                              bbuf0.at[pl.ds(0, TAIL)], semb0)
        ca.wait()
        cb.wait()
        _dot_chunk(abuf0, bbuf0, lg_v, off, TAIL)

    pltpu.sync_copy(lg_v, lg.at[pl.ds(base, EP)])


def _logits_call(out, e0, e1):
    return pl.kernel(
        _logits_body,
        out_type=jax.ShapeDtypeStruct((E,), jnp.float32),
        mesh=_mesh(),
        compiler_params=pltpu.CompilerParams(needs_layout_passes=False,
                                             use_tc_tiling_on_sc=False),
        scratch_types=[
            pltpu.VMEM((EP,), jnp.int32),
            pltpu.VMEM((EP,), jnp.int32),
            pltpu.VMEM((CH, D // 2), jnp.int32),
            pltpu.VMEM((CH, D // 2), jnp.int32),
            pltpu.VMEM((CH, D // 2), jnp.int32),
            pltpu.VMEM((CH, D // 2), jnp.int32),
            pltpu.VMEM((EP,), jnp.float32),
            pltpu.SemaphoreType.DMA,
            pltpu.SemaphoreType.DMA,
            pltpu.SemaphoreType.DMA,
            pltpu.SemaphoreType.DMA,
        ],
    )(out, e0, e1)


def kernel(x_input, edge_index_input, pos_edge_index_input,
           W_in, b_in, W1, att_src1, att_dst1, bias1):
    x = x_input.astype(jnp.float32)
    ei = edge_index_input.astype(jnp.int32)
    pe = pos_edge_index_input.astype(jnp.int32)
    att2 = jnp.stack([att_src1.astype(jnp.float32),
                      att_dst1.astype(jnp.float32)], axis=1)
    x_pad = jnp.pad(x, ((0, NP - N), (0, 0)))
    perm = []
    for w in range(4):
        perm.extend(range(32 * w, 32 * w + 32, 2))
        perm.extend(range(32 * w + 1, 32 * w + 32, 2))
    perm = jnp.array(perm, dtype=jnp.int32)
    w1p = W1[perm, :]
    bias1p = bias1[perm]
    hb, hp, asrc, adst = _prep_call(x_pad, W_in, b_in.reshape(1, D), W1,
                                    w1p, att2)
    hb_i32 = jax.lax.bitcast_convert_type(
        hb.reshape(NP, D // 2, 2), jnp.int32)
    s_part, u_part = _edge_call(asrc, adst, pe[0], pe[1], hb_i32)
    out = _comb_call(s_part.reshape(NW, NP), asrc, adst, u_part, hp,
                     bias1p.reshape(1, D))
    out_i32 = jax.lax.bitcast_convert_type(
        out.reshape(NP, D // 2, 2), jnp.int32)
    return _logits_call(out_i32, ei[0], ei[1])


# speedup vs baseline: 1.1152x; 1.1152x over previous
import jax
import jax.numpy as jnp
from jax import lax
from jax.experimental import pallas as pl
from jax.experimental.pallas import tpu as pltpu
from jax.experimental.pallas import tpu_sc as plsc

N = 10000
NP = 10240
D = 128
E = 320000
NC = 2
NS = 16
NW = NC * NS
EP = E // NW
ROWS = NP // NS
BN = 1024
CH = 32
NCHUNK = EP // CH
TAIL = EP - NCHUNK * CH
CHE = 16
SEG = 2000


def _mesh():
    return plsc.VectorSubcoreMesh(
        core_axis_name="c", subcore_axis_name="s", num_cores=NC, num_subcores=NS
    )


def _pack_bf16_halves(v):
    b = v.astype(jnp.bfloat16)
    lo = lax.bitcast_convert_type(b[:, :64], jnp.uint16).astype(jnp.uint32)
    hi = lax.bitcast_convert_type(b[:, 64:], jnp.uint16).astype(jnp.uint32)
    return lax.bitcast_convert_type(lo | (hi << 16), jnp.int32)


def _prep_body(x_ref, win_ref, b_ref, w1_ref, att2_ref,
               h_ref, hb_ref, asrc_ref, adst_ref):
    t = lax.dot_general(x_ref[...], win_ref[...], (((1,), (1,)), ((), ())),
                        preferred_element_type=jnp.float32) + b_ref[...]
    h = lax.dot_general(t, w1_ref[...], (((1,), (1,)), ((), ())),
                        preferred_element_type=jnp.float32)
    h_ref[...] = h
    hb_ref[...] = _pack_bf16_halves(h)
    att2 = att2_ref[...]
    asrc_ref[...] = lax.dot_general(h, att2[:, 0], (((1,), (0,)), ((), ())),
                                    preferred_element_type=jnp.float32)
    adst_ref[...] = lax.dot_general(h, att2[:, 1], (((1,), (0,)), ((), ())),
                                    preferred_element_type=jnp.float32)


def _prep_call(x, w_in, b_in, w1, att2):
    return pl.pallas_call(
        _prep_body,
        grid=(NP // BN,),
        in_specs=[
            pl.BlockSpec((BN, D), lambda i: (i, 0)),
            pl.BlockSpec((D, D), lambda i: (0, 0)),
            pl.BlockSpec((1, D), lambda i: (0, 0)),
            pl.BlockSpec((D, D), lambda i: (0, 0)),
            pl.BlockSpec((D, 2), lambda i: (0, 0)),
        ],
        out_specs=[
            pl.BlockSpec((BN, D), lambda i: (i, 0)),
            pl.BlockSpec((BN, D // 2), lambda i: (i, 0)),
            pl.BlockSpec((BN,), lambda i: (i,)),
            pl.BlockSpec((BN,), lambda i: (i,)),
        ],
        out_shape=[
            jax.ShapeDtypeStruct((NP, D), jnp.float32),
            jax.ShapeDtypeStruct((NP, D // 2), jnp.int32),
            jax.ShapeDtypeStruct((NP,), jnp.float32),
            jax.ShapeDtypeStruct((NP,), jnp.float32),
        ],
    )(x, w_in, b_in, w1, att2)


def _edge_scale(src_v, dst_v, asrc_v, adst_v, s_loc, hbuf, obuf, off):
    o2 = pl.ds(off, 16)
    d16 = dst_v[o2]
    s16 = src_v[o2]
    av = plsc.load_gather(asrc_v, [s16])
    bv = plsc.load_gather(adst_v, [d16])
    e = av + bv
    e = jnp.where(e >= 0, e, 0.2 * e)
    ex = jnp.exp(e)
    plsc.addupdate_scatter(s_loc, [d16], ex)
    for j in range(16):
        exs = ex[j]
        for w in range(4):
            a2 = plsc.bitcast(hbuf[j, pl.ds(w * 16, 16)], jnp.bfloat16)
            he, ho = plsc.unpack(a2, format=plsc.PackFormat.INTERLEAVED)
            obuf[j, pl.ds(w * 16, 16)] = he * exs
            obuf[j, pl.ds(64 + w * 16, 16)] = ho * exs
    return d16


def _edge_body(asrch, adsth, srch, dsth, hh, s_out, u_out,
               asrc_v, adst_v, src_v, dst_v, s_loc, hbuf0, hbuf1,
               obuf0, obuf1, u_sh, sem0, sem1, sems0, sems1):
    cid = lax.axis_index("c")
    sid = lax.axis_index("s")
    wid = sid * NC + cid
    base = wid * EP

    pltpu.sync_copy(asrch.at[pl.ds(0, N)], asrc_v)
    pltpu.sync_copy(adsth.at[pl.ds(0, N)], adst_v)

    zf = jnp.zeros((16,), jnp.float32)

    def zloop(i, carry):
        s_loc[pl.ds(i * 16, 16)] = zf
        return carry

    lax.fori_loop(0, N // 16, zloop, 0)

    for r in range(CHE):
        for q in range(8):
            obuf0[r, pl.ds(q * 16, 16)] = zf

    def zuloop(i, carry):
        pltpu.sync_copy(obuf0, u_sh.at[pl.ds(sid * ROWS + i * CHE, CHE)])
        return carry

    lax.fori_loop(0, ROWS // CHE, zuloop, 0)
    plsc.subcore_barrier()

    zidx = jnp.zeros((16,), jnp.int32)

    def wait_gather(off, buf, sem):
        pltpu.make_async_copy(hh.at[src_v.at[pl.ds(off, CHE)]],
                              buf, sem).wait()

    def wait_scatter(buf, sem):
        pltpu.make_async_copy(buf, u_sh.at[zidx], sem).wait()


    def seg_loop(si, carry):
        sbase = base + si * SEG
        pltpu.sync_copy(srch.at[pl.ds(sbase, SEG)], src_v)
        pltpu.sync_copy(dsth.at[pl.ds(sbase, SEG)], dst_v)
        pltpu.async_copy(hh.at[src_v.at[pl.ds(0, CHE)]], hbuf0, sem0)

        def pair(io, c2):
            offa = io * (2 * CHE)
            offb = offa + CHE
            offc = offb + CHE
            pltpu.async_copy(hh.at[src_v.at[pl.ds(offb, CHE)]], hbuf1, sem1)
            wait_gather(offa, hbuf0, sem0)

            @pl.when(io > 0)
            def _():
                wait_scatter(obuf0, sems0)

            d16a = _edge_scale(src_v, dst_v, asrc_v, adst_v, s_loc, hbuf0,
                               obuf0, offa)
            pltpu.async_copy(obuf0, u_sh.at[d16a], sems0, add=True)
            pltpu.async_copy(hh.at[src_v.at[pl.ds(offc, CHE)]], hbuf0, sem0)
            wait_gather(offb, hbuf1, sem1)

            @pl.when(io > 0)
            def _():
                wait_scatter(obuf1, sems1)

            d16b = _edge_scale(src_v, dst_v, asrc_v, adst_v, s_loc, hbuf1,
                               obuf1, offb)
            pltpu.async_copy(obuf1, u_sh.at[d16b], sems1, add=True)
            return c2

        lax.fori_loop(0, (SEG // CHE) // 2, pair, 0)
        offe = SEG - CHE
        wait_gather(offe, hbuf0, sem0)
        wait_scatter(obuf0, sems0)
        d16e = _edge_scale(src_v, dst_v, asrc_v, adst_v, s_loc, hbuf0,
                           obuf0, offe)
        wait_scatter(obuf1, sems1)
        pltpu.sync_copy(obuf0, u_sh.at[d16e], add=True)
        return carry

    lax.fori_loop(0, EP // SEG, seg_loop, 0)

    pltpu.sync_copy(s_loc, s_out.at[pl.ds(wid * NP, N)])
    plsc.subcore_barrier()
    pltpu.sync_copy(u_sh.at[pl.ds(sid * ROWS, ROWS)],
                    u_out.at[cid, pl.ds(sid * ROWS, ROWS)])


def _edge_call(asrc, adst, src, dst, h):
    return pl.kernel(
        _edge_body,
        out_type=(
            jax.ShapeDtypeStruct((NW * NP,), jnp.float32),
            jax.ShapeDtypeStruct((NC, NP, D), jnp.float32),
        ),
        mesh=_mesh(),
        compiler_params=pltpu.CompilerParams(needs_layout_passes=False,
                                             use_tc_tiling_on_sc=False),
        scratch_types=[
            pltpu.VMEM((N,), jnp.float32),
            pltpu.VMEM((N,), jnp.float32),
            pltpu.VMEM((SEG,), jnp.int32),
            pltpu.VMEM((SEG,), jnp.int32),
            pltpu.VMEM((N,), jnp.float32),
            pltpu.VMEM((CHE, D // 2), jnp.int32),
            pltpu.VMEM((CHE, D // 2), jnp.int32),
            pltpu.VMEM((CHE, D), jnp.float32),
            pltpu.VMEM((CHE, D), jnp.float32),
            pltpu.VMEM_SHARED((NP, D), jnp.float32),
            pltpu.SemaphoreType.DMA,
            pltpu.SemaphoreType.DMA,
            pltpu.SemaphoreType.DMA,
            pltpu.SemaphoreType.DMA,
        ],
    )(asrc, adst, src, dst, h)


def _comb_body(s_ref, asrc_ref, adst_ref, u_ref, h_ref, b_ref, out_ref):
    sv = jnp.sum(s_ref[...], axis=0)
    al = asrc_ref[...] + adst_ref[...]
    al = jnp.where(al >= 0, al, 0.2 * al)
    al = jnp.exp(al)
    r = 1.0 / (sv + al + 1e-16)
    out_ref[...] = _pack_bf16_halves(
        (u_ref[0] + u_ref[1] + al[:, None] * h_ref[...])
        * r[:, None] + b_ref[...])


def _comb_call(s_part, asrc, adst, u_part, h, bias):
    return pl.pallas_call(
        _comb_body,
        grid=(NP // BN,),
        in_specs=[
            pl.BlockSpec((NW, BN), lambda i: (0, i)),
            pl.BlockSpec((BN,), lambda i: (i,)),
            pl.BlockSpec((BN,), lambda i: (i,)),
            pl.BlockSpec((NC, BN, D), lambda i: (0, i, 0)),
            pl.BlockSpec((BN, D), lambda i: (i, 0)),
            pl.BlockSpec((1, D), lambda i: (0, 0)),
        ],
        out_specs=pl.BlockSpec((BN, D // 2), lambda i: (i, 0)),
        out_shape=jax.ShapeDtypeStruct((NP, D // 2), jnp.int32),
    )(s_part, asrc, adst, u_part, h, bias)


def _edge_dot(abuf, bbuf, row):
    acc = None
    for w in range(4):
        cs = pl.ds(w * 16, 16)
        a2 = plsc.bitcast(abuf[row, cs], jnp.bfloat16)
        b2 = plsc.bitcast(bbuf[row, cs], jnp.bfloat16)
        ae, ao = plsc.unpack(a2, format=plsc.PackFormat.INTERLEAVED)
        be, bo = plsc.unpack(b2, format=plsc.PackFormat.INTERLEAVED)
        t = ae * be + ao * bo
        acc = t if acc is None else acc + t
    return acc


def _dot_chunk(abuf, bbuf, lg_v, off, n_edges):
    lane = lax.iota(jnp.int32, 16)
    for g in range(n_edges // 16):
        dots = jnp.zeros((16,), jnp.float32)
        for j in range(16):
            acc = _edge_dot(abuf, bbuf, g * 16 + j)
            dots = jnp.where(lane == j, jnp.sum(acc), dots)
        lg_v[pl.ds(off + g * 16, 16)] = dots


def _logits_body(outh, e0h, e1h, lg,
                 ia_v, ib_v, abuf0, bbuf0, abuf1, bbuf1, lg_v,
                 sema0, semb0, sema1, semb1):
    cid = lax.axis_index("c")
    sid = lax.axis_index("s")
    wid = sid * NC + cid
    base = wid * EP
    pltpu.sync_copy(e0h.at[pl.ds(base, EP)], ia_v)
    pltpu.sync_copy(e1h.at[pl.ds(base, EP)], ib_v)

    def issue(off, ab, bb, sa, sb):
        pltpu.async_copy(outh.at[ia_v.at[pl.ds(off, CH)]], ab, sa)
        pltpu.async_copy(outh.at[ib_v.at[pl.ds(off, CH)]], bb, sb)

    def drain(off, ab, bb, sa, sb):
        pltpu.make_async_copy(outh.at[ia_v.at[pl.ds(off, CH)]], ab, sa).wait()
        pltpu.make_async_copy(outh.at[ib_v.at[pl.ds(off, CH)]], bb, sb).wait()

    issue(0, abuf0, bbuf0, sema0, semb0)

    def pair(io, carry):
        offa = io * (2 * CH)
        offb = offa + CH
        offc = offb + CH
        issue(offb, abuf1, bbuf1, sema1, semb1)
        drain(offa, abuf0, bbuf0, sema0, semb0)
        _dot_chunk(abuf0, bbuf0, lg_v, offa, CH)

        @pl.when(offc + CH <= EP - TAIL)
        def _():
            issue(offc, abuf0, bbuf0, sema0, semb0)

        drain(offb, abuf1, bbuf1, sema1, semb1)
        _dot_chunk(abuf1, bbuf1, lg_v, offb, CH)
        return carry

    lax.fori_loop(0, NCHUNK // 2, pair, 0)
    if TAIL:
        off = NCHUNK * CH
        ca = pltpu.async_copy(outh.at[ia_v.at[pl.ds(off, TAIL)]],
                              abuf0.at[pl.ds(0, TAIL)], sema0)
        cb = pltpu.async_copy(outh.at[ib_v.at[pl.ds(off, TAIL)]],
                              bbuf0.at[pl.ds(0, TAIL)], semb0)
        ca.wait()
        cb.wait()
        _dot_chunk(abuf0, bbuf0, lg_v, off, TAIL)

    pltpu.sync_copy(lg_v, lg.at[pl.ds(base, EP)])


def _logits_call(out, e0, e1):
    return pl.kernel(
        _logits_body,
        out_type=jax.ShapeDtypeStruct((E,), jnp.float32),
        mesh=_mesh(),
        compiler_params=pltpu.CompilerParams(needs_layout_passes=False,
                                             use_tc_tiling_on_sc=False),
        scratch_types=[
            pltpu.VMEM((EP,), jnp.int32),
            pltpu.VMEM((EP,), jnp.int32),
            pltpu.VMEM((CH, D // 2), jnp.int32),
            pltpu.VMEM((CH, D // 2), jnp.int32),
            pltpu.VMEM((CH, D // 2), jnp.int32),
            pltpu.VMEM((CH, D // 2), jnp.int32),
            pltpu.VMEM((EP,), jnp.float32),
            pltpu.SemaphoreType.DMA,
            pltpu.SemaphoreType.DMA,
            pltpu.SemaphoreType.DMA,
            pltpu.SemaphoreType.DMA,
        ],
    )(out, e0, e1)


def kernel(x_input, edge_index_input, pos_edge_index_input,
           W_in, b_in, W1, att_src1, att_dst1, bias1):
    x = x_input.astype(jnp.float32)
    ei = edge_index_input.astype(jnp.int32)
    pe = pos_edge_index_input.astype(jnp.int32)
    att2 = jnp.stack([att_src1.astype(jnp.float32),
                      att_dst1.astype(jnp.float32)], axis=1)
    x_pad = jnp.pad(x, ((0, NP - N), (0, 0)))
    h, hb32, asrc, adst = _prep_call(x_pad, W_in, b_in.reshape(1, D), W1,
                                     att2)
    s_part, u_part = _edge_call(asrc, adst, pe[0], pe[1], hb32)
    out32 = _comb_call(s_part.reshape(NW, NP), asrc, adst, u_part, h,
                       bias1.reshape(1, D))
    return _logits_call(out32, ei[0], ei[1])


# speedup vs baseline: 1.2168x; 1.0911x over previous
import jax
import jax.numpy as jnp
from jax import lax
from jax.experimental import pallas as pl
from jax.experimental.pallas import tpu as pltpu
from jax.experimental.pallas import tpu_sc as plsc

N = 10000
NP = 10240
D = 128
E = 320000
NC = 2
NS = 16
NW = NC * NS
EP = E // NW
ROWS = NP // NS
BN = 1024
CH = 32
NCHUNK = EP // CH
TAIL = EP - NCHUNK * CH
CHE = 16
SEG = 400
NBUF = 5


def _mesh():
    return plsc.VectorSubcoreMesh(
        core_axis_name="c", subcore_axis_name="s", num_cores=NC, num_subcores=NS
    )


def _pack_bf16_halves(v):
    b = v.astype(jnp.bfloat16)
    lo = lax.bitcast_convert_type(b[:, :64], jnp.uint16).astype(jnp.uint32)
    hi = lax.bitcast_convert_type(b[:, 64:], jnp.uint16).astype(jnp.uint32)
    return lax.bitcast_convert_type(lo | (hi << 16), jnp.int32)


def _prep_body(x_ref, win_ref, b_ref, w1_ref, att2_ref,
               h_ref, hb_ref, asrc_ref, adst_ref):
    t = lax.dot_general(x_ref[...], win_ref[...], (((1,), (1,)), ((), ())),
                        preferred_element_type=jnp.float32) + b_ref[...]
    h = lax.dot_general(t, w1_ref[...], (((1,), (1,)), ((), ())),
                        preferred_element_type=jnp.float32)
    h_ref[...] = h
    hb_ref[...] = _pack_bf16_halves(h)
    att2 = att2_ref[...]
    asrc_ref[...] = lax.dot_general(h, att2[:, 0], (((1,), (0,)), ((), ())),
                                    preferred_element_type=jnp.float32)
    adst_ref[...] = lax.dot_general(h, att2[:, 1], (((1,), (0,)), ((), ())),
                                    preferred_element_type=jnp.float32)


def _prep_call(x, w_in, b_in, w1, att2):
    return pl.pallas_call(
        _prep_body,
        grid=(NP // BN,),
        in_specs=[
            pl.BlockSpec((BN, D), lambda i: (i, 0)),
            pl.BlockSpec((D, D), lambda i: (0, 0)),
            pl.BlockSpec((1, D), lambda i: (0, 0)),
            pl.BlockSpec((D, D), lambda i: (0, 0)),
            pl.BlockSpec((D, 2), lambda i: (0, 0)),
        ],
        out_specs=[
            pl.BlockSpec((BN, D), lambda i: (i, 0)),
            pl.BlockSpec((BN, D // 2), lambda i: (i, 0)),
            pl.BlockSpec((BN,), lambda i: (i,)),
            pl.BlockSpec((BN,), lambda i: (i,)),
        ],
        out_shape=[
            jax.ShapeDtypeStruct((NP, D), jnp.float32),
            jax.ShapeDtypeStruct((NP, D // 2), jnp.int32),
            jax.ShapeDtypeStruct((NP,), jnp.float32),
            jax.ShapeDtypeStruct((NP,), jnp.float32),
        ],
    )(x, w_in, b_in, w1, att2)


def _edge_scale(src_v, dst_v, asrc_v, adst_v, s_loc, hbuf, obuf, off):
    o2 = pl.ds(off, 16)
    d16 = dst_v[o2]
    s16 = src_v[o2]
    av = plsc.load_gather(asrc_v, [s16])
    bv = plsc.load_gather(adst_v, [d16])
    e = av + bv
    e = jnp.where(e >= 0, e, 0.2 * e)
    ex = jnp.exp(e)
    plsc.addupdate_scatter(s_loc, [d16], ex)
    for j in range(16):
        exs = ex[j]
        for w in range(4):
            a2 = plsc.bitcast(hbuf[j, pl.ds(w * 16, 16)], jnp.bfloat16)
            he, ho = plsc.unpack(a2, format=plsc.PackFormat.INTERLEAVED)
            obuf[j, pl.ds(w * 16, 16)] = he * exs
            obuf[j, pl.ds(64 + w * 16, 16)] = ho * exs
    return d16


def _edge_body(asrch, adsth, srch, dsth, hh, s_out, u_out,
               asrc_v, adst_v, src_v, dst_v, s_loc,
               hb0, hb1, hb2, hb3, hb4, ob0, ob1, ob2, ob3, ob4, u_sh,
               g0, g1, g2, g3, g4, s0, s1, s2, s3, s4):
    hbufs = [hb0, hb1, hb2, hb3, hb4]
    obufs = [ob0, ob1, ob2, ob3, ob4]
    gsems = [g0, g1, g2, g3, g4]
    ssems = [s0, s1, s2, s3, s4]
    cid = lax.axis_index("c")
    sid = lax.axis_index("s")
    wid = sid * NC + cid
    base = wid * EP

    pltpu.sync_copy(asrch.at[pl.ds(0, N)], asrc_v)
    pltpu.sync_copy(adsth.at[pl.ds(0, N)], adst_v)

    zf = jnp.zeros((16,), jnp.float32)

    def zloop(i, carry):
        s_loc[pl.ds(i * 16, 16)] = zf
        return carry

    lax.fori_loop(0, N // 16, zloop, 0)

    for r in range(CHE):
        for q in range(8):
            obufs[0][r, pl.ds(q * 16, 16)] = zf

    def zuloop(i, carry):
        pltpu.sync_copy(obufs[0],
                        u_sh.at[pl.ds(sid * ROWS + i * CHE, CHE)])
        return carry

    lax.fori_loop(0, ROWS // CHE, zuloop, 0)
    plsc.subcore_barrier()

    zidx = jnp.zeros((16,), jnp.int32)

    def wait_gather(off, buf, sem):
        pltpu.make_async_copy(hh.at[src_v.at[pl.ds(off, CHE)]],
                              buf, sem).wait()

    def wait_scatter(buf, sem):
        pltpu.make_async_copy(buf, u_sh.at[zidx], sem).wait()

    NPH = SEG // CHE
    NIT = NPH // NBUF

    def seg_loop(si, carry):
        sbase = base + si * SEG
        pltpu.sync_copy(srch.at[pl.ds(sbase, SEG)], src_v)
        pltpu.sync_copy(dsth.at[pl.ds(sbase, SEG)], dst_v)
        pltpu.async_copy(hh.at[src_v.at[pl.ds(0, CHE)]], hbufs[0], gsems[0])
        pltpu.async_copy(hh.at[src_v.at[pl.ds(CHE, CHE)]], hbufs[1], gsems[1])

        def quint(io, c2):
            for p in range(NBUF):
                off = (io * NBUF + p) * CHE
                off2 = off + 2 * CHE
                b2 = (p + 2) % NBUF

                @pl.when(off2 < SEG)
                def _():
                    pltpu.async_copy(hh.at[src_v.at[pl.ds(off2, CHE)]],
                                     hbufs[b2], gsems[b2])

                wait_gather(off, hbufs[p], gsems[p])

                @pl.when((si > 0) | (io > 0))
                def _():
                    wait_scatter(obufs[p], ssems[p])

                d16 = _edge_scale(src_v, dst_v, asrc_v, adst_v, s_loc,
                                  hbufs[p], obufs[p], off)
                pltpu.async_copy(obufs[p], u_sh.at[d16], ssems[p], add=True)
            return c2

        lax.fori_loop(0, NIT, quint, 0)
        return carry

    lax.fori_loop(0, EP // SEG, seg_loop, 0)
    for p in range(NBUF):
        wait_scatter(obufs[p], ssems[p])

    pltpu.sync_copy(s_loc, s_out.at[pl.ds(wid * NP, N)])
    plsc.subcore_barrier()
    pltpu.sync_copy(u_sh.at[pl.ds(sid * ROWS, ROWS)],
                    u_out.at[cid, pl.ds(sid * ROWS, ROWS)])


def _edge_call(asrc, adst, src, dst, h):
    return pl.kernel(
        _edge_body,
        out_type=(
            jax.ShapeDtypeStruct((NW * NP,), jnp.float32),
            jax.ShapeDtypeStruct((NC, NP, D), jnp.float32),
        ),
        mesh=_mesh(),
        compiler_params=pltpu.CompilerParams(needs_layout_passes=False,
                                             use_tc_tiling_on_sc=False),
        scratch_types=[
            pltpu.VMEM((N,), jnp.float32),
            pltpu.VMEM((N,), jnp.float32),
            pltpu.VMEM((SEG,), jnp.int32),
            pltpu.VMEM((SEG,), jnp.int32),
            pltpu.VMEM((N,), jnp.float32),
        ] + [pltpu.VMEM((CHE, D // 2), jnp.int32) for _ in range(5)]
          + [pltpu.VMEM((CHE, D), jnp.float32) for _ in range(5)]
          + [
            pltpu.VMEM_SHARED((NP, D), jnp.float32),
        ] + [pltpu.SemaphoreType.DMA for _ in range(10)],
    )(asrc, adst, src, dst, h)


def _comb_body(s_ref, asrc_ref, adst_ref, u_ref, h_ref, b_ref, out_ref):
    sv = jnp.sum(s_ref[...], axis=0)
    al = asrc_ref[...] + adst_ref[...]
    al = jnp.where(al >= 0, al, 0.2 * al)
    al = jnp.exp(al)
    r = 1.0 / (sv + al + 1e-16)
    out_ref[...] = _pack_bf16_halves(
        (u_ref[0] + u_ref[1] + al[:, None] * h_ref[...])
        * r[:, None] + b_ref[...])


def _comb_call(s_part, asrc, adst, u_part, h, bias):
    return pl.pallas_call(
        _comb_body,
        grid=(NP // BN,),
        in_specs=[
            pl.BlockSpec((NW, BN), lambda i: (0, i)),
            pl.BlockSpec((BN,), lambda i: (i,)),
            pl.BlockSpec((BN,), lambda i: (i,)),
            pl.BlockSpec((NC, BN, D), lambda i: (0, i, 0)),
            pl.BlockSpec((BN, D), lambda i: (i, 0)),
            pl.BlockSpec((1, D), lambda i: (0, 0)),
        ],
        out_specs=pl.BlockSpec((BN, D // 2), lambda i: (i, 0)),
        out_shape=jax.ShapeDtypeStruct((NP, D // 2), jnp.int32),
    )(s_part, asrc, adst, u_part, h, bias)


def _edge_dot(abuf, bbuf, row):
    acc = None
    for w in range(4):
        cs = pl.ds(w * 16, 16)
        a2 = plsc.bitcast(abuf[row, cs], jnp.bfloat16)
        b2 = plsc.bitcast(bbuf[row, cs], jnp.bfloat16)
        ae, ao = plsc.unpack(a2, format=plsc.PackFormat.INTERLEAVED)
        be, bo = plsc.unpack(b2, format=plsc.PackFormat.INTERLEAVED)
        t = ae * be + ao * bo
        acc = t if acc is None else acc + t
    return acc


def _dot_chunk(abuf, bbuf, lg_v, off, n_edges):
    lane = lax.iota(jnp.int32, 16)
    for g in range(n_edges // 16):
        dots = jnp.zeros((16,), jnp.float32)
        for j in range(16):
            acc = _edge_dot(abuf, bbuf, g * 16 + j)
            dots = jnp.where(lane == j, jnp.sum(acc), dots)
        lg_v[pl.ds(off + g * 16, 16)] = dots


def _logits_body(outh, e0h, e1h, lg,
                 ia_v, ib_v, abuf0, bbuf0, abuf1, bbuf1, lg_v,
                 sema0, semb0, sema1, semb1):
    cid = lax.axis_index("c")
    sid = lax.axis_index("s")
    wid = sid * NC + cid
    base = wid * EP
    pltpu.sync_copy(e0h.at[pl.ds(base, EP)], ia_v)
    pltpu.sync_copy(e1h.at[pl.ds(base, EP)], ib_v)

    def issue(off, ab, bb, sa, sb):
        pltpu.async_copy(outh.at[ia_v.at[pl.ds(off, CH)]], ab, sa)
        pltpu.async_copy(outh.at[ib_v.at[pl.ds(off, CH)]], bb, sb)

    def drain(off, ab, bb, sa, sb):
        pltpu.make_async_copy(outh.at[ia_v.at[pl.ds(off, CH)]], ab, sa).wait()
        pltpu.make_async_copy(outh.at[ib_v.at[pl.ds(off, CH)]], bb, sb).wait()

    issue(0, abuf0, bbuf0, sema0, semb0)

    def pair(io, carry):
        offa = io * (2 * CH)
        offb = offa + CH
        offc = offb + CH
        issue(offb, abuf1, bbuf1, sema1, semb1)
        drain(offa, abuf0, bbuf0, sema0, semb0)
        _dot_chunk(abuf0, bbuf0, lg_v, offa, CH)

        @pl.when(offc + CH <= EP - TAIL)
        def _():
            issue(offc, abuf0, bbuf0, sema0, semb0)

        drain(offb, abuf1, bbuf1, sema1, semb1)
        _dot_chunk(abuf1, bbuf1, lg_v, offb, CH)
        return carry

    lax.fori_loop(0, NCHUNK // 2, pair, 0)
    if TAIL:
        off = NCHUNK * CH
        ca = pltpu.async_copy(outh.at[ia_v.at[pl.ds(off, TAIL)]],
                              abuf0.at[pl.ds(0, TAIL)], sema0)
        cb = pltpu.async_copy(outh.at[ib_v.at[pl.ds(off, TAIL)]],
                              bbuf0.at[pl.ds(0, TAIL)], semb0)
        ca.wait()
        cb.wait()
        _dot_chunk(abuf0, bbuf0, lg_v, off, TAIL)

    pltpu.sync_copy(lg_v, lg.at[pl.ds(base, EP)])


def _logits_call(out, e0, e1):
    return pl.kernel(
        _logits_body,
        out_type=jax.ShapeDtypeStruct((E,), jnp.float32),
        mesh=_mesh(),
        compiler_params=pltpu.CompilerParams(needs_layout_passes=False,
                                             use_tc_tiling_on_sc=False),
        scratch_types=[
            pltpu.VMEM((EP,), jnp.int32),
            pltpu.VMEM((EP,), jnp.int32),
            pltpu.VMEM((CH, D // 2), jnp.int32),
            pltpu.VMEM((CH, D // 2), jnp.int32),
            pltpu.VMEM((CH, D // 2), jnp.int32),
            pltpu.VMEM((CH, D // 2), jnp.int32),
            pltpu.VMEM((EP,), jnp.float32),
            pltpu.SemaphoreType.DMA,
            pltpu.SemaphoreType.DMA,
            pltpu.SemaphoreType.DMA,
            pltpu.SemaphoreType.DMA,
        ],
    )(out, e0, e1)


def kernel(x_input, edge_index_input, pos_edge_index_input,
           W_in, b_in, W1, att_src1, att_dst1, bias1):
    x = x_input.astype(jnp.float32)
    ei = edge_index_input.astype(jnp.int32)
    pe = pos_edge_index_input.astype(jnp.int32)
    att2 = jnp.stack([att_src1.astype(jnp.float32),
                      att_dst1.astype(jnp.float32)], axis=1)
    x_pad = jnp.pad(x, ((0, NP - N), (0, 0)))
    h, hb32, asrc, adst = _prep_call(x_pad, W_in, b_in.reshape(1, D), W1,
                                     att2)
    s_part, u_part = _edge_call(asrc, adst, pe[0], pe[1], hb32)
    out32 = _comb_call(s_part.reshape(NW, NP), asrc, adst, u_part, h,
                       bias1.reshape(1, D))
    return _logits_call(out32, ei[0], ei[1])
